# baseline (speedup 1.0000x reference)
import functools

import jax
import jax.numpy as jnp
from jax import lax
from jax.experimental import pallas as pl
from jax.experimental.pallas import tpu as pltpu
from jax.experimental.pallas import tpu_sc as plsc

_NC = 2
_NS = 16
_L = 16

_N = 50000
_E = 800000
_RT = 3136
_RC = 1568
_NPAD = _NS * _RT
_RLAST = _N - (_NS - 1) * _RT - _RC
_EW = _E // (_NC * _NS)
_C = 1000
_NCHUNK = _EW // _C


def _sc_body(a_hbm, b_hbm, x_hbm, edges_hbm, nv_hbm, st_hbm, wtab_hbm,
             av, bv, xv, wrow, nvf, idxc, r3, obf, sem0):
  cid = lax.axis_index("c")
  sid = lax.axis_index("s")
  wid = sid * _NC + cid
  lane = lax.iota(jnp.int32, _L)

  def build_nv(row_base, nrows):
    pltpu.sync_copy(a_hbm.at[pl.ds(row_base * 9, nrows * 9)],
                    av.at[pl.ds(0, nrows * 9)])
    pltpu.sync_copy(b_hbm.at[pl.ds(row_base * 3, nrows * 3)],
                    bv.at[pl.ds(0, nrows * 3)])
    pltpu.sync_copy(x_hbm.at[pl.ds(row_base * 3, nrows * 3)],
                    xv.at[pl.ds(0, nrows * 3)])

    def group(g, carry):
      nid = g * _L + lane
      xs = [plsc.load_gather(xv, [nid * 3 + j]) for j in range(3)]
      for i in range(3):
        bi = plsc.load_gather(bv, [nid * 3 + i])
        plsc.store_scatter(wrow, [nid, jnp.full((_L,), 9 + i, jnp.int32)], bi)
        acc = bi
        for j in range(3):
          aij = plsc.load_gather(av, [nid * 9 + 3 * i + j])
          plsc.store_scatter(
              wrow, [nid, jnp.full((_L,), 3 * i + j, jnp.int32)], aij)
          acc = acc + aij * xs[j]
        plsc.store_scatter(nvf, [nid * 3 + i], acc)
      return carry

    lax.fori_loop(0, nrows // _L, group, 0)
    pltpu.sync_copy(wrow.at[pl.ds(0, nrows)], wtab_hbm.at[pl.ds(row_base, nrows)])
    pltpu.sync_copy(nvf.at[pl.ds(0, nrows * 3)],
                    nv_hbm.at[pl.ds(row_base * 3, nrows * 3)])

  build_nv(sid * _RT, _RC)

  @pl.when(sid < _NS - 1)
  def _():
    build_nv(sid * _RT + _RC, _RC)

  @pl.when(sid == _NS - 1)
  def _():
    build_nv(sid * _RT + _RC, _RLAST)

  plsc.subcore_barrier()

  ebase = wid * _EW
  msk = lane < 12
  operm = jnp.where(lane < 9, lane + lane // 3,
                    jnp.where(lane < 12, 4 * lane - 33, 0))

  def chunk(k, carry):
    cb = ebase + k * _C
    pltpu.sync_copy(edges_hbm.at[pl.ds(cb * 2, _C * 2)], idxc)
    pltpu.async_copy(wtab_hbm.at[idxc], r3, sem0).wait()

    def row(c, rcarry):
      d = r3[2 * c] - r3[2 * c + 1]
      plsc.store_scatter(obf, [c * 12 + operm], d * d, mask=msk)
      return rcarry

    lax.fori_loop(0, _C, row, 0)
    pltpu.sync_copy(obf, st_hbm.at[pl.ds(cb * 12, _C * 12)])
    return carry

  lax.fori_loop(0, _NCHUNK, chunk, 0)


_sc_kernel = functools.partial(
    pl.kernel,
    out_type=(
        jax.ShapeDtypeStruct((_N * 3,), jnp.float32),
        jax.ShapeDtypeStruct((_E * 12,), jnp.float32),
        jax.ShapeDtypeStruct((_NPAD, 16), jnp.float32),
    ),
    mesh=plsc.VectorSubcoreMesh(
        core_axis_name="c", subcore_axis_name="s",
        num_cores=_NC, num_subcores=_NS),
    compiler_params=pltpu.CompilerParams(
        needs_layout_passes=False, use_tc_tiling_on_sc=False),
    scratch_types=[
        pltpu.VMEM((_RC * 9,), jnp.float32),
        pltpu.VMEM((_RC * 3,), jnp.float32),
        pltpu.VMEM((_RC * 3,), jnp.float32),
        pltpu.VMEM((_RC, 16), jnp.float32),
        pltpu.VMEM((_RC * 3,), jnp.float32),
        pltpu.VMEM((_C * 2,), jnp.int32),
        pltpu.VMEM((_C * 2, 16), jnp.float32),
        pltpu.VMEM((_C * 12,), jnp.float32),
        pltpu.SemaphoreType.DMA,
    ],
)(_sc_body)


def kernel(x, edges, A, b):
  B, N, _ = x.shape
  E = edges.shape[0]
  nv, st, _unused_tab = _sc_kernel(
      A.reshape(N * 9), b.reshape(N * 3), x.reshape(N * 3),
      edges.astype(jnp.int32).reshape(E * 2))
  return (nv.reshape(B, N, 3), st.reshape(B, E, 3, 4))


# speedup vs baseline: 4.6671x; 4.6671x over previous
import functools

import jax
import jax.numpy as jnp
from jax import lax
from jax.experimental import pallas as pl
from jax.experimental.pallas import tpu as pltpu
from jax.experimental.pallas import tpu_sc as plsc

_NC = 2
_NS = 16
_NW = _NC * _NS
_L = 16

_N = 50000
_E = 800000
_RT = 3136
_RC = 1568
_NPAD = _NS * _RT
_RLAST = _N - (_NS - 1) * _RT - _RC

_NBLK = _E // 128
_CB = 8
_NQ = -(-_NBLK // _CB)
_CBT = _NBLK - (_NQ - 1) * _CB


def _sc_body(a_hbm, b_hbm, x_hbm, e_hbm, nv_hbm, st_hbm, wtab_hbm,
             av, bv, xv, wrow, nvf, idxc, r3, obf, obft, sem0):
  cid = lax.axis_index("c")
  sid = lax.axis_index("s")
  wid = sid * _NC + cid
  lane = lax.iota(jnp.int32, _L)

  def build_nv(row_base, nrows):
    pltpu.sync_copy(a_hbm.at[pl.ds(row_base * 9, nrows * 9)],
                    av.at[pl.ds(0, nrows * 9)])
    pltpu.sync_copy(b_hbm.at[pl.ds(row_base * 3, nrows * 3)],
                    bv.at[pl.ds(0, nrows * 3)])
    pltpu.sync_copy(x_hbm.at[pl.ds(row_base * 3, nrows * 3)],
                    xv.at[pl.ds(0, nrows * 3)])

    def group(g, carry):
      nid = g * _L + lane
      xs = [plsc.load_gather(xv, [nid * 3 + j]) for j in range(3)]
      for i in range(3):
        bi = plsc.load_gather(bv, [nid * 3 + i])
        plsc.store_scatter(wrow, [nid, jnp.full((_L,), 9 + i, jnp.int32)], bi)
        acc = bi
        for j in range(3):
          aij = plsc.load_gather(av, [nid * 9 + 3 * i + j])
          plsc.store_scatter(
              wrow, [nid, jnp.full((_L,), 3 * i + j, jnp.int32)], aij)
          acc = acc + aij * xs[j]
        plsc.store_scatter(nvf, [nid * 3 + i], acc)
      return carry

    lax.fori_loop(0, nrows // _L, group, 0)
    pltpu.sync_copy(wrow.at[pl.ds(0, nrows)],
                    wtab_hbm.at[pl.ds(row_base, nrows)])
    pltpu.sync_copy(nvf.at[pl.ds(0, nrows * 3)],
                    nv_hbm.at[pl.ds(row_base * 3, nrows * 3)])

  build_nv(sid * _RT, _RC)

  @pl.when(sid < _NS - 1)
  def _():
    build_nv(sid * _RT + _RC, _RC)

  @pl.when(sid == _NS - 1)
  def _():
    build_nv(sid * _RT + _RC, _RLAST)

  plsc.subcore_barrier()

  msk = lane < 12
  comp_i = jnp.where(msk, jnp.where(lane < 9, lane // 3, lane - 9), 0)
  comp_j = jnp.where(lane < 9, lane - (lane // 3) * 3, 3)

  def do_chunk(q, nblk, obuf):
    nedge = nblk * 128
    pltpu.sync_copy(e_hbm.at[pl.ds(q * (_CB * 256), nedge * 2)],
                    idxc.at[pl.ds(0, nedge * 2)])
    pltpu.async_copy(wtab_hbm.at[idxc.at[pl.ds(0, nedge * 2)]],
                     r3.at[pl.ds(0, nedge * 2)], sem0).wait()

    def col(ep, carry):
      ep_v = lane * 0 + ep
      for blk in range(nblk):
        d = r3[blk * 256 + ep] - r3[blk * 256 + 128 + ep]
        plsc.store_scatter(
            obuf, [comp_i, jnp.full((_L,), blk, jnp.int32), comp_j, ep_v],
            d * d, mask=msk)
      return carry

    lax.fori_loop(0, 128, col, 0)
    pltpu.sync_copy(obuf, st_hbm.at[:, pl.ds(q * _CB, nblk)])

  def chunk(k, carry):
    q = wid + _NW * k

    @pl.when(q < _NQ - 1)
    def _():
      do_chunk(q, _CB, obf)

    @pl.when(q == _NQ - 1)
    def _():
      do_chunk(q, _CBT, obft)

    return carry

  nk = (_NQ - wid + _NW - 1) // _NW
  lax.fori_loop(0, nk, chunk, 0)


_sc_kernel = functools.partial(
    pl.kernel,
    out_type=(
        jax.ShapeDtypeStruct((_N * 3,), jnp.float32),
        jax.ShapeDtypeStruct((3, _NBLK, 4, 128), jnp.float32),
        jax.ShapeDtypeStruct((_NPAD, 16), jnp.float32),
    ),
    mesh=plsc.VectorSubcoreMesh(
        core_axis_name="c", subcore_axis_name="s",
        num_cores=_NC, num_subcores=_NS),
    compiler_params=pltpu.CompilerParams(
        needs_layout_passes=False, use_tc_tiling_on_sc=False),
    scratch_types=[
        pltpu.VMEM((_RC * 9,), jnp.float32),
        pltpu.VMEM((_RC * 3,), jnp.float32),
        pltpu.VMEM((_RC * 3,), jnp.float32),
        pltpu.VMEM((_RC, 16), jnp.float32),
        pltpu.VMEM((_RC * 3,), jnp.float32),
        pltpu.VMEM((_CB * 256,), jnp.int32),
        pltpu.VMEM((_CB * 256, 16), jnp.float32),
        pltpu.VMEM((3, _CB, 4, 128), jnp.float32),
        pltpu.VMEM((3, _CBT, 4, 128), jnp.float32),
        pltpu.SemaphoreType.DMA,
    ],
)(_sc_body)


def kernel(x, edges, A, b):
  B, N, _ = x.shape
  E = edges.shape[0]
  ev = edges.astype(jnp.int32).reshape(_NBLK, 128, 2)
  ev = ev.transpose(0, 2, 1).reshape(_NBLK * 256)
  nv, st, _unused_tab = _sc_kernel(
      A.reshape(N * 9), b.reshape(N * 3), x.reshape(N * 3), ev)
  stiffness = st.transpose(1, 3, 0, 2).reshape(B, E, 3, 4)
  return (nv.reshape(B, N, 3), stiffness)
